# baseline (speedup 1.0000x reference)
import functools

import jax
import jax.numpy as jnp
from jax import lax
from jax.experimental import pallas as pl
from jax.experimental.pallas import tpu as pltpu
from jax.experimental.pallas import tpu_sc as plsc

T, D, H, E = 4096, 1024, 4096, 8
NC, NS = 2, 16
NW = NC * NS
TPW = T // NW
MT = 256
M = 2 * T + E * MT
NMT = M // MT
NMT_PAD = 48
HB = 512

_sc_params = pltpu.CompilerParams(needs_layout_passes=False)


def _router_body(x_ref, wg_ref, i0_ref, i1_ref, r0_ref, r1_ref,
                 w0_ref, w1_ref, cnt_ref, eid_ref, carry_ref):
    t = pl.program_id(0)

    @pl.when(t == 0)
    def _():
        carry_ref[...] = jnp.zeros_like(carry_ref)

    logits = jnp.dot(x_ref[...], wg_ref[...], preferred_element_type=jnp.float32)
    probs = jax.nn.softmax(logits, axis=-1)
    iota = lax.broadcasted_iota(jnp.int32, (TPW, E), 1)
    m0 = jnp.max(probs, axis=1, keepdims=True)
    i0 = jnp.argmax(probs, axis=1)[:, None]
    h0 = iota == i0
    masked = jnp.where(h0, -jnp.inf, probs)
    m1 = jnp.max(masked, axis=1, keepdims=True)
    i1 = jnp.argmax(masked, axis=1)[:, None]
    h1 = iota == i1
    s = m0 + m1 + 1e-9

    ind = h0.astype(jnp.float32) + h1.astype(jnp.float32)
    rr = lax.broadcasted_iota(jnp.int32, (TPW, TPW), 0)
    cc = lax.broadcasted_iota(jnp.int32, (TPW, TPW), 1)
    tri = (rr > cc).astype(jnp.float32)
    cumex = jnp.dot(tri, ind, preferred_element_type=jnp.float32)
    tot = cumex + carry_ref[...]
    rank0 = jnp.sum(jnp.where(h0, tot, 0.0), axis=1)
    rank1 = jnp.sum(jnp.where(h1, tot, 0.0), axis=1)

    i0_ref[...] = i0[:, 0].astype(jnp.int32).reshape(1, 1, TPW)
    i1_ref[...] = i1[:, 0].astype(jnp.int32).reshape(1, 1, TPW)
    r0_ref[...] = rank0.astype(jnp.int32).reshape(1, 1, TPW)
    r1_ref[...] = rank1.astype(jnp.int32).reshape(1, 1, TPW)
    w0_ref[...] = (m0[:, 0] / s[:, 0]).reshape(1, 1, TPW)
    w1_ref[...] = (m1[:, 0] / s[:, 0]).reshape(1, 1, TPW)

    carry = carry_ref[...] + jnp.sum(ind, axis=0, keepdims=True)
    carry_ref[...] = carry
    cnt_ref[...] = jnp.concatenate(
        [carry, jnp.zeros_like(carry)], axis=1).astype(jnp.int32).reshape(1, 1, 2 * E)

    padded = jnp.ceil(carry / MT) * MT
    ee = lax.broadcasted_iota(jnp.int32, (E, E), 0)
    ff = lax.broadcasted_iota(jnp.int32, (E, E), 1)
    tri8 = (ee < ff).astype(jnp.float32)
    off = jnp.dot(padded, tri8, preferred_element_type=jnp.float32)
    pos = lax.broadcasted_iota(jnp.int32, (NMT_PAD, E), 0).astype(jnp.float32) * MT
    eid = jnp.sum((pos >= off).astype(jnp.int32), axis=1) - 1
    eid_ref[...] = eid.reshape(1, 1, NMT_PAD)


def _router(x, Wg):
    i3 = lambda sh, dt: jax.ShapeDtypeStruct(sh, dt)
    return pl.pallas_call(
        _router_body,
        grid=(NW,),
        in_specs=[
            pl.BlockSpec((TPW, D), lambda t: (t, 0)),
            pl.BlockSpec((D, E), lambda t: (0, 0)),
        ],
        out_specs=[
            pl.BlockSpec((1, 1, TPW), lambda t: (t, 0, 0)),
            pl.BlockSpec((1, 1, TPW), lambda t: (t, 0, 0)),
            pl.BlockSpec((1, 1, TPW), lambda t: (t, 0, 0)),
            pl.BlockSpec((1, 1, TPW), lambda t: (t, 0, 0)),
            pl.BlockSpec((1, 1, TPW), lambda t: (t, 0, 0)),
            pl.BlockSpec((1, 1, TPW), lambda t: (t, 0, 0)),
            pl.BlockSpec((1, 1, 2 * E), lambda t: (0, 0, 0)),
            pl.BlockSpec((1, 1, NMT_PAD), lambda t: (0, 0, 0)),
        ],
        out_shape=[
            i3((NW, 1, TPW), jnp.int32), i3((NW, 1, TPW), jnp.int32),
            i3((NW, 1, TPW), jnp.int32), i3((NW, 1, TPW), jnp.int32),
            i3((NW, 1, TPW), jnp.float32), i3((NW, 1, TPW), jnp.float32),
            i3((1, 1, 2 * E), jnp.int32),
            i3((1, 1, NMT_PAD), jnp.int32),
        ],
        scratch_shapes=[pltpu.VMEM((1, E), jnp.float32)],
        compiler_params=pltpu.CompilerParams(dimension_semantics=("arbitrary",)),
    )(x, Wg)


@functools.cache
def _make_dispatch():
    return functools.partial(
        pl.kernel,
        out_type=(
            jax.ShapeDtypeStruct((M, D), jnp.float32),
            jax.ShapeDtypeStruct((NW, 1, TPW), jnp.int32),
            jax.ShapeDtypeStruct((NW, 1, TPW), jnp.int32),
        ),
        mesh=plsc.VectorSubcoreMesh(core_axis_name="c", subcore_axis_name="s"),
        scratch_types=[
        pltpu.VMEM((16,), jnp.int32),
        pltpu.VMEM((16,), jnp.int32),
        pltpu.VMEM((1, TPW), jnp.int32),
        pltpu.VMEM((1, TPW), jnp.int32),
        pltpu.VMEM((1, TPW), jnp.int32),
        pltpu.VMEM((1, TPW), jnp.int32),
        pltpu.VMEM((1, TPW), jnp.int32),
        pltpu.VMEM((1, TPW), jnp.int32),
        pltpu.VMEM((32, D), jnp.float32),
        pltpu.VMEM((32, D), jnp.float32),
        pltpu.VMEM((32,), jnp.int32),
        pltpu.VMEM((32,), jnp.int32),
        pltpu.VMEM((32,), jnp.int32),
        pltpu.VMEM((32,), jnp.int32),
        pltpu.SemaphoreType.DMA,
        pltpu.SemaphoreType.DMA,
        pltpu.SemaphoreType.DMA,
        pltpu.SemaphoreType.DMA,
        ],
        compiler_params=_sc_params,
    )(_dispatch_body)


def _dispatch_body(x_hbm, i0_hbm, i1_hbm, r0_hbm, r1_hbm, cnt_hbm,
              xs_hbm, d0_hbm, d1_hbm,
              cntv, offv, i0v, i1v, r0v, r1v, d0v, d1v,
              xr0, xr1, diA0, diB0, diA1, diB1,
              semx0, semx1, sems0, sems1):
    wid = lax.axis_index("s") * NC + lax.axis_index("c")
    base = wid * TPW

    pltpu.sync_copy(cnt_hbm.at[0, 0], cntv)
    cnt = cntv[...]
    padded = jnp.bitwise_and(cnt + (MT - 1), jnp.int32(-MT))
    incl = plsc.cumsum(padded)
    offv[...] = incl - padded

    pltpu.sync_copy(i0_hbm.at[wid], i0v)
    pltpu.sync_copy(i1_hbm.at[wid], i1v)
    pltpu.sync_copy(r0_hbm.at[wid], r0v)
    pltpu.sync_copy(r1_hbm.at[wid], r1v)

    for j in range(TPW // 16):
        sl = pl.ds(16 * j, 16)
        off0 = plsc.load_gather(offv, [i0v[0, sl]])
        off1 = plsc.load_gather(offv, [i1v[0, sl]])
        d0v[0, sl] = off0 + r0v[0, sl]
        d1v[0, sl] = off1 + r1v[0, sl]
    pltpu.sync_copy(d0v, d0_hbm.at[wid])
    pltpu.sync_copy(d1v, d1_hbm.at[wid])

    xr = [xr0, xr1]
    diA = [diA0, diA1]
    diB = [diB0, diB1]
    semx = [semx0, semx1]
    sems = [sems0, sems1]
    NSUB = TPW // 32
    xh = {0: pltpu.async_copy(x_hbm.at[pl.ds(base, 32)], xr[0], semx[0])}
    sh = {}
    for s in range(NSUB):
        p = s % 2
        if s >= 1:
            sh[s - 1][0].wait()
            sh[s - 1][1].wait()
        if s + 1 < NSUB:
            xh[s + 1] = pltpu.async_copy(
                x_hbm.at[pl.ds(base + 32 * (s + 1), 32)], xr[1 - p],
                semx[1 - p])
        xh[s].wait()
        diA[p][pl.ds(0, 16)] = d0v[0, pl.ds(32 * s, 16)]
        diA[p][pl.ds(16, 16)] = d0v[0, pl.ds(32 * s + 16, 16)]
        diB[p][pl.ds(0, 16)] = d1v[0, pl.ds(32 * s, 16)]
        diB[p][pl.ds(16, 16)] = d1v[0, pl.ds(32 * s + 16, 16)]
        sh[s] = (pltpu.async_copy(xr[p], xs_hbm.at[diA[p]], sems[p]),
                 pltpu.async_copy(xr[p], xs_hbm.at[diB[p]], sems[p]))
    sh[NSUB - 1][0].wait()
    sh[NSUB - 1][1].wait()


HH = H // 2


def _gemm_body(eid_ref, xs_ref, w1_ref, w2_ref, os_ref, hb_s):
    h = pl.program_id(1)
    xb = xs_ref[...].astype(jnp.bfloat16)
    for hb in range(HH // HB):
        sl = pl.ds(hb * HB, HB)
        hb_s[:, sl] = jax.nn.gelu(jnp.dot(xb, w1_ref[0, :, sl],
                                          preferred_element_type=jnp.float32)
                                  ).astype(jnp.bfloat16)
    part = jnp.dot(hb_s[...], w2_ref[0], preferred_element_type=jnp.float32)

    @pl.when(h == 0)
    def _():
        os_ref[...] = part

    @pl.when(h == 1)
    def _():
        os_ref[...] += part


def _grouped_gemm(eid, xs, w1, w2):
    grid_spec = pltpu.PrefetchScalarGridSpec(
        num_scalar_prefetch=1,
        grid=(NMT, 2),
        in_specs=[
            pl.BlockSpec((MT, D), lambda m, h, eid_ref: (m, 0)),
            pl.BlockSpec((1, D, HH), lambda m, h, eid_ref: (eid_ref[m], 0, h)),
            pl.BlockSpec((1, HH, D), lambda m, h, eid_ref: (eid_ref[m], h, 0)),
        ],
        out_specs=pl.BlockSpec((MT, D), lambda m, h, eid_ref: (m, 0)),
        scratch_shapes=[pltpu.VMEM((MT, HH), jnp.bfloat16)],
    )
    return pl.pallas_call(
        _gemm_body,
        grid_spec=grid_spec,
        out_shape=jax.ShapeDtypeStruct((M, D), jnp.float32),
        compiler_params=pltpu.CompilerParams(
            dimension_semantics=("arbitrary", "arbitrary")),
    )(eid, xs, w1, w2)


@functools.cache
def _make_combine():
    return functools.partial(
        pl.kernel,
        out_type=jax.ShapeDtypeStruct((T, D), jnp.float32),
        mesh=plsc.VectorSubcoreMesh(core_axis_name="c", subcore_axis_name="s"),
        scratch_types=[
            pltpu.VMEM((1, TPW), jnp.int32),
            pltpu.VMEM((1, TPW), jnp.int32),
            pltpu.VMEM((1, TPW), jnp.float32),
            pltpu.VMEM((1, TPW), jnp.float32),
            pltpu.VMEM((D,), jnp.float32),
            pltpu.VMEM((16,), jnp.int32),
            pltpu.VMEM((16,), jnp.int32),
            pltpu.VMEM((16,), jnp.int32),
            pltpu.VMEM((16,), jnp.int32),
            pltpu.VMEM((16, D), jnp.float32),
            pltpu.VMEM((16, D), jnp.float32),
            pltpu.VMEM((16, D), jnp.float32),
            pltpu.VMEM((16, D), jnp.float32),
            pltpu.VMEM((16, D), jnp.float32),
            pltpu.VMEM((16, D), jnp.float32),
            pltpu.SemaphoreType.DMA,
            pltpu.SemaphoreType.DMA,
            pltpu.SemaphoreType.DMA,
            pltpu.SemaphoreType.DMA,
        ],
        compiler_params=_sc_params,
    )(_combine_body)


def _combine_body(os_hbm, d0_hbm, d1_hbm, w0_hbm, w1_hbm, bias_hbm, out_hbm,
             d0v, d1v, w0v, w1v, biasv, gA0, gB0, gA1, gB1,
             bufA0, bufB0, bufA1, bufB1, outb0, outb1,
             semg0, semg1, semo0, semo1):
    wid = lax.axis_index("s") * NC + lax.axis_index("c")
    base = wid * TPW
    pltpu.sync_copy(d0_hbm.at[wid], d0v)
    pltpu.sync_copy(d1_hbm.at[wid], d1v)
    pltpu.sync_copy(w0_hbm.at[wid], w0v)
    pltpu.sync_copy(w1_hbm.at[wid], w1v)
    pltpu.sync_copy(bias_hbm, biasv)
    z16 = jnp.zeros((16,), jnp.int32)
    gA = [gA0, gA1]
    gB = [gB0, gB1]
    bufA = [bufA0, bufA1]
    bufB = [bufB0, bufB1]
    outb = [outb0, outb1]
    semg = [semg0, semg1]
    semo = [semo0, semo1]
    NS8 = TPW // 16

    def issue(s):
        p = s % 2
        gA[p][...] = d0v[0, pl.ds(16 * s, 16)]
        gB[p][...] = d1v[0, pl.ds(16 * s, 16)]
        hA = pltpu.async_copy(os_hbm.at[gA[p]], bufA[p], semg[p])
        hB = pltpu.async_copy(os_hbm.at[gB[p]], bufB[p], semg[p])
        return (hA, hB)

    gh = {0: issue(0)}
    oh = {}
    for s in range(NS8):
        p = s % 2
        if s + 1 < NS8:
            gh[s + 1] = issue(s + 1)
        gh[s][0].wait()
        gh[s][1].wait()
        if s >= 2:
            oh[s - 2].wait()

        def body(r, carry):
            lane = jnp.full((16,), 16 * s + r, jnp.int32)
            wa = plsc.load_gather(w0v, [z16, lane])
            wb = plsc.load_gather(w1v, [z16, lane])
            for c in range(D // 16):
                sl = pl.ds(16 * c, 16)
                outb[p][r, sl] = (bufA[p][r, sl] * wa
                                  + bufB[p][r, sl] * wb + biasv[sl])
            return carry

        lax.fori_loop(0, 16, body, 0)
        oh[s] = pltpu.async_copy(outb[p], out_hbm.at[pl.ds(base + 16 * s, 16)],
                                 semo[p])
    oh[NS8 - 2].wait()
    oh[NS8 - 1].wait()


def kernel(x, Wg, w1, w2, global_bias):
    i0, i1, r0, r1, w0r, w1r, cnt, eid = _router(x, Wg)
    xs, d0, d1 = _make_dispatch()(x, i0, i1, r0, r1, cnt)
    os = _grouped_gemm(eid.reshape(NMT_PAD), xs, w1, w2)
    return _make_combine()(os, d0, d1, w0r, w1r, global_bias)


# speedup vs baseline: 1.1904x; 1.1904x over previous
import functools

import jax
import jax.numpy as jnp
from jax import lax
from jax.experimental import pallas as pl
from jax.experimental.pallas import tpu as pltpu
from jax.experimental.pallas import tpu_sc as plsc

T, D, H, E = 4096, 1024, 4096, 8
NC, NS = 2, 16
NW = NC * NS
TPW = T // NW
MT = 256
M = 2 * T + E * MT
NMT = M // MT
NMT_PAD = 48
HB = 512

_sc_params = pltpu.CompilerParams(needs_layout_passes=False)


def _router_body(x_ref, wg_ref, i0_ref, i1_ref, r0_ref, r1_ref,
                 w0_ref, w1_ref, cnt_ref, eid_ref, carry_ref):
    t = pl.program_id(0)

    @pl.when(t == 0)
    def _():
        carry_ref[...] = jnp.zeros_like(carry_ref)

    logits = jnp.dot(x_ref[...], wg_ref[...], preferred_element_type=jnp.float32)
    probs = jax.nn.softmax(logits, axis=-1)
    iota = lax.broadcasted_iota(jnp.int32, (TPW, E), 1)
    m0 = jnp.max(probs, axis=1, keepdims=True)
    i0 = jnp.argmax(probs, axis=1)[:, None]
    h0 = iota == i0
    masked = jnp.where(h0, -jnp.inf, probs)
    m1 = jnp.max(masked, axis=1, keepdims=True)
    i1 = jnp.argmax(masked, axis=1)[:, None]
    h1 = iota == i1
    s = m0 + m1 + 1e-9

    ind = h0.astype(jnp.float32) + h1.astype(jnp.float32)
    rr = lax.broadcasted_iota(jnp.int32, (TPW, TPW), 0)
    cc = lax.broadcasted_iota(jnp.int32, (TPW, TPW), 1)
    tri = (rr > cc).astype(jnp.float32)
    cumex = jnp.dot(tri, ind, preferred_element_type=jnp.float32)
    tot = cumex + carry_ref[...]
    rank0 = jnp.sum(jnp.where(h0, tot, 0.0), axis=1)
    rank1 = jnp.sum(jnp.where(h1, tot, 0.0), axis=1)

    i0_ref[...] = i0[:, 0].astype(jnp.int32).reshape(1, 1, TPW)
    i1_ref[...] = i1[:, 0].astype(jnp.int32).reshape(1, 1, TPW)
    r0_ref[...] = rank0.astype(jnp.int32).reshape(1, 1, TPW)
    r1_ref[...] = rank1.astype(jnp.int32).reshape(1, 1, TPW)
    w0_ref[...] = (m0[:, 0] / s[:, 0]).reshape(1, 1, TPW)
    w1_ref[...] = (m1[:, 0] / s[:, 0]).reshape(1, 1, TPW)

    carry = carry_ref[...] + jnp.sum(ind, axis=0, keepdims=True)
    carry_ref[...] = carry
    cnt_ref[...] = jnp.concatenate(
        [carry, jnp.zeros_like(carry)], axis=1).astype(jnp.int32).reshape(1, 1, 2 * E)

    padded = jnp.ceil(carry / MT) * MT
    ee = lax.broadcasted_iota(jnp.int32, (E, E), 0)
    ff = lax.broadcasted_iota(jnp.int32, (E, E), 1)
    tri8 = (ee < ff).astype(jnp.float32)
    off = jnp.dot(padded, tri8, preferred_element_type=jnp.float32)
    pos = lax.broadcasted_iota(jnp.int32, (NMT_PAD, E), 0).astype(jnp.float32) * MT
    eid = jnp.sum((pos >= off).astype(jnp.int32), axis=1) - 1
    eid_ref[...] = eid.reshape(1, 1, NMT_PAD)


def _router(x, Wg):
    i3 = lambda sh, dt: jax.ShapeDtypeStruct(sh, dt)
    return pl.pallas_call(
        _router_body,
        grid=(NW,),
        in_specs=[
            pl.BlockSpec((TPW, D), lambda t: (t, 0)),
            pl.BlockSpec((D, E), lambda t: (0, 0)),
        ],
        out_specs=[
            pl.BlockSpec((1, 1, TPW), lambda t: (t, 0, 0)),
            pl.BlockSpec((1, 1, TPW), lambda t: (t, 0, 0)),
            pl.BlockSpec((1, 1, TPW), lambda t: (t, 0, 0)),
            pl.BlockSpec((1, 1, TPW), lambda t: (t, 0, 0)),
            pl.BlockSpec((1, 1, TPW), lambda t: (t, 0, 0)),
            pl.BlockSpec((1, 1, TPW), lambda t: (t, 0, 0)),
            pl.BlockSpec((1, 1, 2 * E), lambda t: (0, 0, 0)),
            pl.BlockSpec((1, 1, NMT_PAD), lambda t: (0, 0, 0)),
        ],
        out_shape=[
            i3((NW, 1, TPW), jnp.int32), i3((NW, 1, TPW), jnp.int32),
            i3((NW, 1, TPW), jnp.int32), i3((NW, 1, TPW), jnp.int32),
            i3((NW, 1, TPW), jnp.float32), i3((NW, 1, TPW), jnp.float32),
            i3((1, 1, 2 * E), jnp.int32),
            i3((1, 1, NMT_PAD), jnp.int32),
        ],
        scratch_shapes=[pltpu.VMEM((1, E), jnp.float32)],
        compiler_params=pltpu.CompilerParams(dimension_semantics=("arbitrary",)),
    )(x, Wg)


@functools.cache
def _make_dispatch():
    return functools.partial(
        pl.kernel,
        out_type=(
            jax.ShapeDtypeStruct((M, D), jnp.float32),
            jax.ShapeDtypeStruct((NW, 1, TPW), jnp.int32),
            jax.ShapeDtypeStruct((NW, 1, TPW), jnp.int32),
        ),
        mesh=plsc.VectorSubcoreMesh(core_axis_name="c", subcore_axis_name="s"),
        scratch_types=[
        pltpu.VMEM((16,), jnp.int32),
        pltpu.VMEM((16,), jnp.int32),
        pltpu.VMEM((1, TPW), jnp.int32),
        pltpu.VMEM((1, TPW), jnp.int32),
        pltpu.VMEM((1, TPW), jnp.int32),
        pltpu.VMEM((1, TPW), jnp.int32),
        pltpu.VMEM((1, TPW), jnp.int32),
        pltpu.VMEM((1, TPW), jnp.int32),
        pltpu.VMEM((32, D), jnp.float32),
        pltpu.VMEM((32, D), jnp.float32),
        pltpu.VMEM((32,), jnp.int32),
        pltpu.VMEM((32,), jnp.int32),
        pltpu.VMEM((32,), jnp.int32),
        pltpu.VMEM((32,), jnp.int32),
        pltpu.SemaphoreType.DMA,
        pltpu.SemaphoreType.DMA,
        pltpu.SemaphoreType.DMA,
        pltpu.SemaphoreType.DMA,
        ],
        compiler_params=_sc_params,
    )(_dispatch_body)


def _dispatch_body(x_hbm, i0_hbm, i1_hbm, r0_hbm, r1_hbm, cnt_hbm,
              xs_hbm, d0_hbm, d1_hbm,
              cntv, offv, i0v, i1v, r0v, r1v, d0v, d1v,
              xr0, xr1, diA0, diB0, diA1, diB1,
              semx0, semx1, sems0, sems1):
    wid = lax.axis_index("s") * NC + lax.axis_index("c")
    base = wid * TPW

    pltpu.sync_copy(cnt_hbm.at[0, 0], cntv)
    cnt = cntv[...]
    padded = jnp.bitwise_and(cnt + (MT - 1), jnp.int32(-MT))
    incl = plsc.cumsum(padded)
    offv[...] = incl - padded

    pltpu.sync_copy(i0_hbm.at[wid], i0v)
    pltpu.sync_copy(i1_hbm.at[wid], i1v)
    pltpu.sync_copy(r0_hbm.at[wid], r0v)
    pltpu.sync_copy(r1_hbm.at[wid], r1v)

    for j in range(TPW // 16):
        sl = pl.ds(16 * j, 16)
        off0 = plsc.load_gather(offv, [i0v[0, sl]])
        off1 = plsc.load_gather(offv, [i1v[0, sl]])
        d0v[0, sl] = off0 + r0v[0, sl]
        d1v[0, sl] = off1 + r1v[0, sl]
    pltpu.sync_copy(d0v, d0_hbm.at[wid])
    pltpu.sync_copy(d1v, d1_hbm.at[wid])

    xr = [xr0, xr1]
    diA = [diA0, diA1]
    diB = [diB0, diB1]
    semx = [semx0, semx1]
    sems = [sems0, sems1]
    NSUB = TPW // 32
    xh = {0: pltpu.async_copy(x_hbm.at[pl.ds(base, 32)], xr[0], semx[0])}
    sh = {}
    for s in range(NSUB):
        p = s % 2
        if s >= 1:
            sh[s - 1][0].wait()
            sh[s - 1][1].wait()
        if s + 1 < NSUB:
            xh[s + 1] = pltpu.async_copy(
                x_hbm.at[pl.ds(base + 32 * (s + 1), 32)], xr[1 - p],
                semx[1 - p])
        xh[s].wait()
        diA[p][pl.ds(0, 16)] = d0v[0, pl.ds(32 * s, 16)]
        diA[p][pl.ds(16, 16)] = d0v[0, pl.ds(32 * s + 16, 16)]
        diB[p][pl.ds(0, 16)] = d1v[0, pl.ds(32 * s, 16)]
        diB[p][pl.ds(16, 16)] = d1v[0, pl.ds(32 * s + 16, 16)]
        sh[s] = (pltpu.async_copy(xr[p], xs_hbm.at[diA[p]], sems[p]),
                 pltpu.async_copy(xr[p], xs_hbm.at[diB[p]], sems[p]))
    sh[NSUB - 1][0].wait()
    sh[NSUB - 1][1].wait()


def _gemm_body(eid_ref, xs_ref, w1_ref, w2_ref, os_ref, hb_s):
    xb = xs_ref[...].astype(jnp.bfloat16)
    for hb in range(H // HB):
        sl = pl.ds(hb * HB, HB)
        hb_s[:, sl] = jax.nn.gelu(jnp.dot(xb, w1_ref[0, :, sl],
                                          preferred_element_type=jnp.float32)
                                  ).astype(jnp.bfloat16)
    os_ref[...] = jnp.dot(hb_s[...], w2_ref[0],
                          preferred_element_type=jnp.float32)


def _grouped_gemm(eid, xs, w1, w2):
    grid_spec = pltpu.PrefetchScalarGridSpec(
        num_scalar_prefetch=1,
        grid=(NMT,),
        in_specs=[
            pl.BlockSpec((MT, D), lambda m, eid_ref: (m, 0)),
            pl.BlockSpec((1, D, H), lambda m, eid_ref: (eid_ref[m], 0, 0)),
            pl.BlockSpec((1, H, D), lambda m, eid_ref: (eid_ref[m], 0, 0)),
        ],
        out_specs=pl.BlockSpec((MT, D), lambda m, eid_ref: (m, 0)),
        scratch_shapes=[pltpu.VMEM((MT, H), jnp.bfloat16)],
    )
    return pl.pallas_call(
        _gemm_body,
        grid_spec=grid_spec,
        out_shape=jax.ShapeDtypeStruct((M, D), jnp.float32),
        compiler_params=pltpu.CompilerParams(
            dimension_semantics=("arbitrary",)),
    )(eid, xs, w1, w2)


@functools.cache
def _make_combine():
    return functools.partial(
        pl.kernel,
        out_type=jax.ShapeDtypeStruct((T, D), jnp.float32),
        mesh=plsc.VectorSubcoreMesh(core_axis_name="c", subcore_axis_name="s"),
        scratch_types=[
            pltpu.VMEM((1, TPW), jnp.int32),
            pltpu.VMEM((1, TPW), jnp.int32),
            pltpu.VMEM((1, TPW), jnp.float32),
            pltpu.VMEM((1, TPW), jnp.float32),
            pltpu.VMEM((D,), jnp.float32),
            pltpu.VMEM((16,), jnp.int32),
            pltpu.VMEM((16,), jnp.int32),
            pltpu.VMEM((16,), jnp.int32),
            pltpu.VMEM((16,), jnp.int32),
            pltpu.VMEM((16, D), jnp.float32),
            pltpu.VMEM((16, D), jnp.float32),
            pltpu.VMEM((16, D), jnp.float32),
            pltpu.VMEM((16, D), jnp.float32),
            pltpu.VMEM((16, D), jnp.float32),
            pltpu.VMEM((16, D), jnp.float32),
            pltpu.SemaphoreType.DMA,
            pltpu.SemaphoreType.DMA,
            pltpu.SemaphoreType.DMA,
            pltpu.SemaphoreType.DMA,
        ],
        compiler_params=_sc_params,
    )(_combine_body)


def _combine_body(os_hbm, d0_hbm, d1_hbm, w0_hbm, w1_hbm, bias_hbm, out_hbm,
             d0v, d1v, w0v, w1v, biasv, gA0, gB0, gA1, gB1,
             bufA0, bufB0, bufA1, bufB1, outb0, outb1,
             semg0, semg1, semo0, semo1):
    wid = lax.axis_index("s") * NC + lax.axis_index("c")
    base = wid * TPW
    pltpu.sync_copy(d0_hbm.at[wid], d0v)
    pltpu.sync_copy(d1_hbm.at[wid], d1v)
    pltpu.sync_copy(w0_hbm.at[wid], w0v)
    pltpu.sync_copy(w1_hbm.at[wid], w1v)
    pltpu.sync_copy(bias_hbm, biasv)
    z16 = jnp.zeros((16,), jnp.int32)
    gA = [gA0, gA1]
    gB = [gB0, gB1]
    bufA = [bufA0, bufA1]
    bufB = [bufB0, bufB1]
    outb = [outb0, outb1]
    semg = [semg0, semg1]
    semo = [semo0, semo1]
    NS8 = TPW // 16

    def issue(s):
        p = s % 2
        gA[p][...] = d0v[0, pl.ds(16 * s, 16)]
        gB[p][...] = d1v[0, pl.ds(16 * s, 16)]
        hA = pltpu.async_copy(os_hbm.at[gA[p]], bufA[p], semg[p])
        hB = pltpu.async_copy(os_hbm.at[gB[p]], bufB[p], semg[p])
        return (hA, hB)

    gh = {0: issue(0)}
    oh = {}
    for s in range(NS8):
        p = s % 2
        if s + 1 < NS8:
            gh[s + 1] = issue(s + 1)
        gh[s][0].wait()
        gh[s][1].wait()
        if s >= 2:
            oh[s - 2].wait()

        def body(r, carry):
            lane = jnp.full((16,), 16 * s + r, jnp.int32)
            wa = plsc.load_gather(w0v, [z16, lane])
            wb = plsc.load_gather(w1v, [z16, lane])
            for c in range(D // 16):
                sl = pl.ds(16 * c, 16)
                outb[p][r, sl] = (bufA[p][r, sl] * wa
                                  + bufB[p][r, sl] * wb + biasv[sl])
            return carry

        lax.fori_loop(0, 16, body, 0)
        oh[s] = pltpu.async_copy(outb[p], out_hbm.at[pl.ds(base + 16 * s, 16)],
                                 semo[p])
    oh[NS8 - 2].wait()
    oh[NS8 - 1].wait()


def kernel(x, Wg, w1, w2, global_bias):
    i0, i1, r0, r1, w0r, w1r, cnt, eid = _router(x, Wg)
    xs, d0, d1 = _make_dispatch()(x, i0, i1, r0, r1, cnt)
    os = _grouped_gemm(eid.reshape(NMT_PAD), xs,
                       w1.astype(jnp.bfloat16), w2.astype(jnp.bfloat16))
    return _make_combine()(os, d0, d1, w0r, w1r, global_bias)
